# Initial kernel scaffold; baseline (speedup 1.0000x reference)
import jax
import jax.numpy as jnp
from jax.experimental import pallas as pl


def kernel(path, ent_table, rel_table, W1, b1, W2, b2, e, q, rs, es, action_keys, segment_ids):
    raise NotImplementedError("write your pallas kernel here")


# speedup vs baseline: 1.8675x; 1.8675x over previous
import functools

import jax
import jax.numpy as jnp
from jax import lax
from jax.experimental import pallas as pl
from jax.experimental.pallas import tpu as pltpu
from jax.experimental.pallas import tpu_sc as plsc

B = 4096
N_ACT = 131072
N_KEYS = 262144
E_VOCAB = 100000
R_VOCAB = 1000
ENT_DIM = 200
REL_DIM = 200
HIST_DIM = 200

NC = 2
NS = 16
NW = NC * NS
L = 16


def _mesh():
    return plsc.VectorSubcoreMesh(
        core_axis_name="c", subcore_axis_name="s",
        num_cores=NC, num_subcores=NS)


def _wid():
    return lax.axis_index("s") * NC + lax.axis_index("c")


ROWS_PER_W = B // NW


@functools.partial(
    pl.kernel,
    out_type=(jax.ShapeDtypeStruct((B, ENT_DIM), jnp.float32),
              jax.ShapeDtypeStruct((B, REL_DIM), jnp.float32)),
    mesh=_mesh(),
    scratch_types=[
        pltpu.VMEM((ROWS_PER_W,), jnp.int32),
        pltpu.VMEM((ROWS_PER_W,), jnp.int32),
        pltpu.VMEM((ROWS_PER_W, ENT_DIM), jnp.float32),
        pltpu.VMEM((ROWS_PER_W, REL_DIM), jnp.float32),
        pltpu.SemaphoreType.DMA,
    ],
    compiler_params=pltpu.CompilerParams(use_tc_tiling_on_sc=False, needs_layout_passes=False),
)
def _eq_gather(ent_hbm, rel_hbm, e_hbm, q_hbm, eout, qout,
               ei_v, qi_v, e_v, q_v, sem):
    base = _wid() * ROWS_PER_W
    pltpu.sync_copy(e_hbm.at[pl.ds(base, ROWS_PER_W)], ei_v)
    pltpu.sync_copy(q_hbm.at[pl.ds(base, ROWS_PER_W)], qi_v)
    c1 = pltpu.async_copy(ent_hbm.at[ei_v], e_v, sem)
    c2 = pltpu.async_copy(rel_hbm.at[qi_v], q_v, sem)
    c1.wait()
    c2.wait()
    pltpu.sync_copy(e_v, eout.at[pl.ds(base, ROWS_PER_W)])
    pltpu.sync_copy(q_v, qout.at[pl.ds(base, ROWS_PER_W)])


BLK = 256


def _mlp_body(e_ref, h_ref, q_ref, w1_ref, b1_ref, w2a_ref, w2b_ref,
              b2a_ref, b2b_ref, rel_ref, rp_ref, u_ref):
    x = jnp.dot(e_ref[...], w1_ref[0:ENT_DIM, :],
                preferred_element_type=jnp.float32)
    x = x + jnp.dot(h_ref[...], w1_ref[ENT_DIM:ENT_DIM + HIST_DIM, :],
                    preferred_element_type=jnp.float32)
    x = x + jnp.dot(q_ref[...], w1_ref[ENT_DIM + HIST_DIM:, :],
                    preferred_element_type=jnp.float32)
    x = jnp.maximum(x + b1_ref[...], 0.0)
    x2a = jnp.dot(x, w2a_ref[...], preferred_element_type=jnp.float32)
    x2a = x2a + b2a_ref[...]
    x2b = jnp.dot(x, w2b_ref[...], preferred_element_type=jnp.float32)
    x2b = x2b + b2b_ref[...]
    rp_ref[...] = lax.dot_general(
        x2a, rel_ref[...], (((1,), (1,)), ((), ())),
        preferred_element_type=jnp.float32)
    u_ref[...] = x2b


def _mlp(eemb, h, qemb, W1, b1, W2, b2, rel_table):
    W2a, W2b = W2[:, :REL_DIM], W2[:, REL_DIM:]
    b2a, b2b = b2[:REL_DIM].reshape(1, -1), b2[REL_DIM:].reshape(1, -1)
    return pl.pallas_call(
        _mlp_body,
        grid=(B // BLK,),
        in_specs=[
            pl.BlockSpec((BLK, ENT_DIM), lambda i: (i, 0)),
            pl.BlockSpec((BLK, HIST_DIM), lambda i: (i, 0)),
            pl.BlockSpec((BLK, REL_DIM), lambda i: (i, 0)),
            pl.BlockSpec((ENT_DIM + HIST_DIM + REL_DIM, 400), lambda i: (0, 0)),
            pl.BlockSpec((1, 400), lambda i: (0, 0)),
            pl.BlockSpec((400, REL_DIM), lambda i: (0, 0)),
            pl.BlockSpec((400, ENT_DIM), lambda i: (0, 0)),
            pl.BlockSpec((1, REL_DIM), lambda i: (0, 0)),
            pl.BlockSpec((1, ENT_DIM), lambda i: (0, 0)),
            pl.BlockSpec((R_VOCAB, REL_DIM), lambda i: (0, 0)),
        ],
        out_specs=[
            pl.BlockSpec((BLK, R_VOCAB), lambda i: (i, 0)),
            pl.BlockSpec((BLK, ENT_DIM), lambda i: (i, 0)),
        ],
        out_shape=[
            jax.ShapeDtypeStruct((B, R_VOCAB), jnp.float32),
            jax.ShapeDtypeStruct((B, ENT_DIM), jnp.float32),
        ],
    )(eemb, h, qemb, W1, b1.reshape(1, -1), W2a, W2b, b2a, b2b, rel_table)


KPW = N_KEYS // NW
NB = 128
NBLK = KPW // NB


@functools.partial(
    pl.kernel,
    out_type=jax.ShapeDtypeStruct((N_KEYS,), jnp.float32),
    mesh=_mesh(),
    scratch_types=[
        pltpu.VMEM((NB,), jnp.int32),
        pltpu.VMEM((NB,), jnp.int32),
        pltpu.VMEM((NB,), jnp.int32),
        pltpu.VMEM((NB,), jnp.int32),
        pltpu.VMEM((NB,), jnp.int32),
        pltpu.VMEM((NB,), jnp.float32),
        pltpu.VMEM((NB, ENT_DIM), jnp.float32),
        pltpu.VMEM((NB, ENT_DIM), jnp.float32),
        pltpu.VMEM((NB,), jnp.float32),
        pltpu.SemaphoreType.DMA,
    ],
    compiler_params=pltpu.CompilerParams(use_tc_tiling_on_sc=False, needs_layout_passes=False),
)
def _score(rp_hbm, u_hbm, ent_hbm, rs_hbm, es_hbm, ak_hbm, seg_hbm, out_hbm,
           ak_v, seg_v, rs_v, es_v, rpi_v, rp_v, ent_v, u_v, sc_v, sem):
    base0 = _wid() * KPW

    def blk(bi, carry):
        base = base0 + bi * NB
        pltpu.sync_copy(ak_hbm.at[pl.ds(base, NB)], ak_v)
        pltpu.sync_copy(seg_hbm.at[pl.ds(base, NB)], seg_v)
        c1 = pltpu.async_copy(rs_hbm.at[ak_v], rs_v, sem)
        c2 = pltpu.async_copy(es_hbm.at[ak_v], es_v, sem)
        c1.wait()
        c2.wait()
        for g in range(NB // L):
            s16 = seg_v[pl.ds(g * L, L)]
            r16 = rs_v[pl.ds(g * L, L)]
            rpi_v[pl.ds(g * L, L)] = s16 * R_VOCAB + r16
        c3 = pltpu.async_copy(rp_hbm.at[rpi_v], rp_v, sem)
        c4 = pltpu.async_copy(ent_hbm.at[es_v], ent_v, sem)
        c5 = pltpu.async_copy(u_hbm.at[seg_v], u_v, sem)
        c3.wait()
        c4.wait()
        c5.wait()
        for g in range(NB // L):
            row = lax.iota(jnp.int32, L) + g * L

            def dstep(j, acc):
                for t in range(8):
                    col = jnp.full((L,), j * 8 + t, jnp.int32)
                    ev = plsc.load_gather(ent_v, [row, col])
                    uv = plsc.load_gather(u_v, [row, col])
                    acc = acc + ev * uv
                return acc

            acc = lax.fori_loop(0, ENT_DIM // 8, dstep,
                                jnp.zeros((L,), jnp.float32))
            sc_v[pl.ds(g * L, L)] = acc + rp_v[pl.ds(g * L, L)]
        pltpu.sync_copy(sc_v, out_hbm.at[pl.ds(base, NB)])
        return carry

    lax.fori_loop(0, NBLK, blk, 0)


def kernel(path, ent_table, rel_table, W1, b1, W2, b2, e, q, rs, es,
           action_keys, segment_ids):
    H = path[0, 0, 2]
    e = e.astype(jnp.int32)
    q = q.astype(jnp.int32)
    rs = rs.astype(jnp.int32)
    es = es.astype(jnp.int32)
    action_keys = action_keys.astype(jnp.int32)
    segment_ids = segment_ids.astype(jnp.int32)
    eemb, qemb = _eq_gather(ent_table, rel_table, e, q)
    rp, u = _mlp(eemb, H, qemb, W1, b1, W2, b2, rel_table)
    return _score(rp.reshape(-1), u, ent_table, rs, es,
                  action_keys, segment_ids)
